# Initial kernel scaffold; baseline (speedup 1.0000x reference)
import jax
import jax.numpy as jnp
from jax.experimental import pallas as pl


def kernel(hidden_states, q_weight, k_weight):
    raise NotImplementedError("write your pallas kernel here")


# speedup vs baseline: 12.8739x; 12.8739x over previous
import functools

import jax
import jax.numpy as jnp
from jax.experimental import pallas as pl


def _hnet_kernel(hs_ref, qwT_ref, kwT_ref, out_ref, *, L, D):
    hs = hs_ref[0]
    qwT = qwT_ref[...]
    kwT = kwT_ref[...]

    q = jnp.dot(hs, qwT, preferred_element_type=jnp.float32)
    k = jnp.dot(hs, kwT, preferred_element_type=jnp.float32)

    nq = jnp.maximum(jnp.sqrt(jnp.sum(q * q, axis=1, keepdims=True)), 1e-12)
    nk = jnp.maximum(jnp.sqrt(jnp.sum(k * k, axis=1, keepdims=True)), 1e-12)

    zrow = jnp.zeros((1, D), dtype=jnp.float32)
    k_next = jnp.concatenate([k[1:], zrow], axis=0)
    nk_next = jnp.concatenate([nk[1:], jnp.ones((1, 1), jnp.float32)], axis=0)

    dot_qk = jnp.sum(q * k_next, axis=1, keepdims=True)
    cos = dot_qk / (nq * nk_next)

    pm = jnp.clip((1.0 - cos) * 0.5, 0.0, 1.0)
    prob = jnp.concatenate([jnp.ones((1, 1), jnp.float32), pm[:L - 1]], axis=0)

    mask = prob > 0.5
    a_col = jnp.where(mask, 1.0 - prob, 1.0)
    c_col = jnp.where(mask, prob, 0.0)

    a = jnp.broadcast_to(a_col, (L, D))
    b = c_col * hs

    s = 1
    while s < L:
        b_sh = jnp.concatenate([jnp.zeros((s, D), jnp.float32), b[:L - s]], axis=0)
        a_sh = jnp.concatenate([jnp.ones((s, D), jnp.float32), a[:L - s]], axis=0)
        b = b + a * b_sh
        a = a * a_sh
        s *= 2

    out_ref[0] = hs + b


def kernel(hidden_states, q_weight, k_weight):
    B, L, D = hidden_states.shape
    qwT = q_weight.T
    kwT = k_weight.T
    return pl.pallas_call(
        functools.partial(_hnet_kernel, L=L, D=D),
        grid=(B,),
        in_specs=[
            pl.BlockSpec((1, L, D), lambda b: (b, 0, 0)),
            pl.BlockSpec((D, D), lambda b: (0, 0)),
            pl.BlockSpec((D, D), lambda b: (0, 0)),
        ],
        out_specs=pl.BlockSpec((1, L, D), lambda b: (b, 0, 0)),
        out_shape=jax.ShapeDtypeStruct((B, L, D), hidden_states.dtype),
    )(hidden_states, qwT, kwT)


# speedup vs baseline: 12.9832x; 1.0085x over previous
import functools

import jax
import jax.numpy as jnp
from jax.experimental import pallas as pl


def _hnet_kernel(hs_ref, qwT_ref, kwT_ref, out_ref, *, L, D):
    hs = hs_ref[0]
    qwT = qwT_ref[...]
    kwT = kwT_ref[...]

    q = jnp.dot(hs, qwT, preferred_element_type=jnp.float32)
    k = jnp.dot(hs, kwT, preferred_element_type=jnp.float32)

    nq = jnp.maximum(jnp.sqrt(jnp.sum(q * q, axis=1, keepdims=True)), 1e-12)
    nk = jnp.maximum(jnp.sqrt(jnp.sum(k * k, axis=1, keepdims=True)), 1e-12)

    zrow = jnp.zeros((1, D), dtype=jnp.float32)
    k_next = jnp.concatenate([k[1:], zrow], axis=0)
    nk_next = jnp.concatenate([nk[1:], jnp.ones((1, 1), jnp.float32)], axis=0)

    dot_qk = jnp.sum(q * k_next, axis=1, keepdims=True)
    cos = dot_qk / (nq * nk_next)

    pm = jnp.clip((1.0 - cos) * 0.5, 0.0, 1.0)
    prob = jnp.concatenate([jnp.ones((1, 1), jnp.float32), pm[:L - 1]], axis=0)

    mask = prob > 0.5
    a_col = jnp.where(mask, 1.0 - prob, 1.0)
    c_col = jnp.where(mask, prob, 0.0)

    W = 128
    NB = D // W
    a = jnp.broadcast_to(a_col, (L, W))
    c = jnp.broadcast_to(c_col, (L, W))
    bs = [c * hs[:, j * W:(j + 1) * W] for j in range(NB)]

    s = 1
    while s < L:
        zpad = jnp.zeros((s, W), jnp.float32)
        a_sh = jnp.concatenate([jnp.ones((s, W), jnp.float32), a[:L - s]], axis=0)
        bs = [b + a * jnp.concatenate([zpad, b[:L - s]], axis=0) for b in bs]
        a = a * a_sh
        s *= 2

    for j in range(NB):
        out_ref[0, :, j * W:(j + 1) * W] = hs[:, j * W:(j + 1) * W] + bs[j]


def kernel(hidden_states, q_weight, k_weight):
    B, L, D = hidden_states.shape
    qwT = q_weight.T
    kwT = k_weight.T
    return pl.pallas_call(
        functools.partial(_hnet_kernel, L=L, D=D),
        grid=(B,),
        in_specs=[
            pl.BlockSpec((1, L, D), lambda b: (b, 0, 0)),
            pl.BlockSpec((D, D), lambda b: (0, 0)),
            pl.BlockSpec((D, D), lambda b: (0, 0)),
        ],
        out_specs=pl.BlockSpec((1, L, D), lambda b: (b, 0, 0)),
        out_shape=jax.ShapeDtypeStruct((B, L, D), hidden_states.dtype),
    )(hidden_states, qwT, kwT)
